# Initial kernel scaffold; baseline (speedup 1.0000x reference)
import jax
import jax.numpy as jnp
from jax.experimental import pallas as pl


def kernel(node_h, edge_index, edge_h, mW1, mb1, mW2, mb2, Wih, Whh, bih, bhh, gamma, beta, eW1, eb1, eW2, eb2):
    raise NotImplementedError("write your pallas kernel here")


# speedup vs baseline: 1.4756x; 1.4756x over previous
import functools

import jax
import jax.numpy as jnp
from jax import lax
from jax.experimental import pallas as pl
from jax.experimental.pallas import tpu as pltpu
from jax.experimental.pallas import tpu_sc as plsc

N = 10000
E = 320000
D = 128
ED = 16

_NC = 2
_NS = 16
_L = 16
_NW = _NC * _NS

_CH = 128
_NCHUNKS = E // _CH
_CHW = _NCHUNKS // _NW
_CHREM = _NCHUNKS % _NW
_NP = 10240
_RPT = _NP // _NS
_ZR = 128

_PREC = jax.lax.Precision.HIGHEST

_mesh = plsc.VectorSubcoreMesh(core_axis_name="c", subcore_axis_name="s")
_mesh1 = plsc.VectorSubcoreMesh(core_axis_name="c", subcore_axis_name="s",
                                num_cores=1)
_CHW1 = _NCHUNKS // _NS
_CHREM1 = _NCHUNKS % _NS


def _node_pre_body(x_ref, w_ref, b_ref, p_ref, q_ref, g_ref):
    y = jnp.dot(x_ref[...], w_ref[...], preferred_element_type=jnp.float32,
                precision=_PREC) + b_ref[...]
    p_ref[...] = y[:, :D]
    q_ref[...] = y[:, D:2 * D]
    g_ref[...] = y[:, 2 * D:]


def _node_pre(node_h, Wn, bn):
    BN = 2000
    return pl.pallas_call(
        _node_pre_body,
        grid=(N // BN,),
        in_specs=[
            pl.BlockSpec((BN, D), lambda i: (i, 0)),
            pl.BlockSpec((D, 5 * D), lambda i: (0, 0)),
            pl.BlockSpec((1, 5 * D), lambda i: (0, 0)),
        ],
        out_specs=[
            pl.BlockSpec((BN, D), lambda i: (i, 0)),
            pl.BlockSpec((BN, D), lambda i: (i, 0)),
            pl.BlockSpec((BN, 3 * D), lambda i: (i, 0)),
        ],
        out_shape=[
            jax.ShapeDtypeStruct((N, D), jnp.float32),
            jax.ShapeDtypeStruct((N, D), jnp.float32),
            jax.ShapeDtypeStruct((N, 3 * D), jnp.float32),
        ],
    )(node_h, Wn, bn)


def _edge_pre_body(e_ref, w_ref, b_ref, r_ref):
    r_ref[...] = jnp.dot(e_ref[...], w_ref[...],
                         preferred_element_type=jnp.float32,
                         precision=_PREC) + b_ref[...]


def _edge_pre(edge_h, W1eT, mb1):
    BE = 2000
    return pl.pallas_call(
        _edge_pre_body,
        grid=(E // BE,),
        in_specs=[
            pl.BlockSpec((BE, ED), lambda i: (i, 0)),
            pl.BlockSpec((ED, D), lambda i: (0, 0)),
            pl.BlockSpec((1, D), lambda i: (0, 0)),
        ],
        out_specs=pl.BlockSpec((BE, D), lambda i: (i, 0)),
        out_shape=jax.ShapeDtypeStruct((E, D), jnp.float32),
    )(edge_h, W1eT, mb1)


def _node_upd_body(h0_ref, h1_ref, d0_ref, d1_ref, x_ref, gh_ref,
                   mw2_ref, mb2_ref, wih_ref, bih_ref, gam_ref, bet_ref,
                   ew_ref, n_ref, pe_ref, qe_ref):
    hs = h0_ref[...] + h1_ref[...]
    deg = d0_ref[:, 0:1] + d1_ref[:, 0:1]
    agg = (jnp.dot(hs, mw2_ref[...], preferred_element_type=jnp.float32,
                   precision=_PREC) + deg * mb2_ref[...])
    gi = jnp.dot(agg, wih_ref[...], preferred_element_type=jnp.float32,
                 precision=_PREC) + bih_ref[...]
    gh = gh_ref[...]
    x = x_ref[...]
    r = jax.nn.sigmoid(gi[:, :D] + gh[:, :D])
    z = jax.nn.sigmoid(gi[:, D:2 * D] + gh[:, D:2 * D])
    n = jnp.tanh(gi[:, 2 * D:] + r * gh[:, 2 * D:])
    new_h = (1.0 - z) * n + z * x
    mu = jnp.mean(new_h, axis=-1, keepdims=True)
    xc = new_h - mu
    var = jnp.mean(xc * xc, axis=-1, keepdims=True)
    normed = xc * jax.lax.rsqrt(var + 1e-5) * gam_ref[...] + bet_ref[...]
    n_ref[...] = normed
    pq = jnp.dot(normed, ew_ref[...], preferred_element_type=jnp.float32,
                 precision=_PREC)
    pe_ref[...] = pq[:, :D]
    qe_ref[...] = pq[:, D:]


def _node_upd(H0, H1, D0, D1, node_h, GH, mW2T, mb2, WihT, bih, gam, bet, eWst):
    BN = 2000
    return pl.pallas_call(
        _node_upd_body,
        grid=(N // BN,),
        in_specs=[
            pl.BlockSpec((BN, D), lambda i: (i, 0)),
            pl.BlockSpec((BN, D), lambda i: (i, 0)),
            pl.BlockSpec((BN, ED), lambda i: (i, 0)),
            pl.BlockSpec((BN, ED), lambda i: (i, 0)),
            pl.BlockSpec((BN, D), lambda i: (i, 0)),
            pl.BlockSpec((BN, 3 * D), lambda i: (i, 0)),
            pl.BlockSpec((D, D), lambda i: (0, 0)),
            pl.BlockSpec((1, D), lambda i: (0, 0)),
            pl.BlockSpec((D, 3 * D), lambda i: (0, 0)),
            pl.BlockSpec((1, 3 * D), lambda i: (0, 0)),
            pl.BlockSpec((1, D), lambda i: (0, 0)),
            pl.BlockSpec((1, D), lambda i: (0, 0)),
            pl.BlockSpec((D, 2 * D), lambda i: (0, 0)),
        ],
        out_specs=[
            pl.BlockSpec((BN, D), lambda i: (i, 0)),
            pl.BlockSpec((BN, D), lambda i: (i, 0)),
            pl.BlockSpec((BN, D), lambda i: (i, 0)),
        ],
        out_shape=[
            jax.ShapeDtypeStruct((N, D), jnp.float32),
            jax.ShapeDtypeStruct((N, D), jnp.float32),
            jax.ShapeDtypeStruct((N, D), jnp.float32),
        ],
    )(H0, H1, D0, D1, node_h, GH, mW2T, mb2, WihT, bih, gam, bet, eWst)


def _edge_fin_body(s_ref, e_ref, w1_ref, b1_ref, w2_ref, b2_ref, o_ref):
    eh = e_ref[...]
    re = jnp.dot(eh, w1_ref[...], preferred_element_type=jnp.float32,
                 precision=_PREC) + b1_ref[...]
    e1 = jnp.maximum(s_ref[...] + re, 0.0)
    eo = jnp.dot(e1, w2_ref[...], preferred_element_type=jnp.float32,
                 precision=_PREC) + b2_ref[...]
    o_ref[...] = jnp.maximum(eo + eh, 0.0)


def _edge_fin(esum, edge_h, eW1eT, eb1, eW2T, eb2):
    BE = 2000
    return pl.pallas_call(
        _edge_fin_body,
        grid=(E // BE,),
        in_specs=[
            pl.BlockSpec((BE, D), lambda i: (i, 0)),
            pl.BlockSpec((BE, ED), lambda i: (i, 0)),
            pl.BlockSpec((ED, D), lambda i: (0, 0)),
            pl.BlockSpec((1, D), lambda i: (0, 0)),
            pl.BlockSpec((D, ED), lambda i: (0, 0)),
            pl.BlockSpec((1, ED), lambda i: (0, 0)),
        ],
        out_specs=pl.BlockSpec((BE, ED), lambda i: (i, 0)),
        out_shape=jax.ShapeDtypeStruct((E, ED), jnp.float32),
    )(esum, edge_h, eW1eT, eb1, eW2T, eb2)


_HALF = 1280
_NPASS = 10240 // _HALF
_HPAD = _HALF + 32
_ZRH = _HPAD // _NS
_OPT = _HALF // _NS
_TRASH = _HALF + 8


def _sc_wid():
    return lax.axis_index("s") * _NC + lax.axis_index("c")


def _chunk_index(wid, i):
    return jnp.where(i < _CHW, wid * _CHW + i, _NW * _CHW + wid)


def _n_chunks(wid):
    return _CHW + jnp.where(wid < _CHREM, 1, 0)


def _sc_h1(P, Q, R, src, tgt):
    @functools.partial(
        pl.kernel,
        out_type=jax.ShapeDtypeStruct((E, D), jnp.float32),
        mesh=_mesh,
        scratch_types=[
            pltpu.VMEM((_CH,), jnp.int32),
            pltpu.VMEM((_CH,), jnp.int32),
            pltpu.VMEM((_CH, D), jnp.float32),
            pltpu.VMEM((_CH, D), jnp.float32),
            pltpu.VMEM((_CH, D), jnp.float32),
            pltpu.SemaphoreType.DMA,
            pltpu.SemaphoreType.DMA,
            pltpu.SemaphoreType.DMA,
        ],
    )
    def body(p_hbm, q_hbm, r_hbm, src_hbm, tgt_hbm, h1_hbm,
             srcv, tgtv, bufP, bufQ, bufR, semP, semQ, semR):
        wid = _sc_wid()

        def do_chunk(e0):
            pltpu.sync_copy(tgt_hbm.at[pl.ds(e0, _CH)], tgtv)
            pltpu.sync_copy(src_hbm.at[pl.ds(e0, _CH)], srcv)
            cpP = pltpu.async_copy(p_hbm.at[srcv], bufP, semP)
            cpQ = pltpu.async_copy(q_hbm.at[tgtv], bufQ, semQ)
            cpR = pltpu.async_copy(r_hbm.at[pl.ds(e0, _CH)], bufR, semR)
            cpP.wait()
            cpQ.wait()
            cpR.wait()

            @pl.loop(0, _CH)
            def _r(rr):
                @pl.loop(0, D, step=_L)
                def _c(cc):
                    v = (bufP[rr, pl.ds(cc, _L)] + bufQ[rr, pl.ds(cc, _L)]
                         + bufR[rr, pl.ds(cc, _L)])
                    bufP[rr, pl.ds(cc, _L)] = jnp.maximum(v, 0.0)

            pltpu.sync_copy(bufP, h1_hbm.at[pl.ds(e0, _CH)])

        @pl.loop(0, _CHW)
        def _main(i):
            do_chunk((wid * _CHW + i) * _CH)

        @pl.when(wid < _CHREM)
        def _tail():
            do_chunk((_NW * _CHW + wid) * _CH)

    return body(P, Q, R, src, tgt)


def _sc_esum(Pe, Qe, src, tgt):
    @functools.partial(
        pl.kernel,
        out_type=jax.ShapeDtypeStruct((E, D), jnp.float32),
        mesh=_mesh,
        scratch_types=[
            pltpu.VMEM((_CH,), jnp.int32),
            pltpu.VMEM((_CH,), jnp.int32),
            pltpu.VMEM((_CH, D), jnp.float32),
            pltpu.VMEM((_CH, D), jnp.float32),
            pltpu.SemaphoreType.DMA,
            pltpu.SemaphoreType.DMA,
        ],
    )
    def body(p_hbm, q_hbm, src_hbm, tgt_hbm, out_hbm,
             srcv, tgtv, bufP, bufQ, semP, semQ):
        wid = _sc_wid()

        def do_chunk(e0):
                pltpu.sync_copy(src_hbm.at[pl.ds(e0, _CH)], srcv)
                pltpu.sync_copy(tgt_hbm.at[pl.ds(e0, _CH)], tgtv)
                cpP = pltpu.async_copy(p_hbm.at[srcv], bufP, semP)
                cpQ = pltpu.async_copy(q_hbm.at[tgtv], bufQ, semQ)
                cpP.wait()
                cpQ.wait()

                @pl.loop(0, _CH)
                def _r(rr):
                    @pl.loop(0, D, step=_L)
                    def _c(cc):
                        bufP[rr, pl.ds(cc, _L)] = (bufP[rr, pl.ds(cc, _L)]
                                                   + bufQ[rr, pl.ds(cc, _L)])

                pltpu.sync_copy(bufP, out_hbm.at[pl.ds(e0, _CH)])

        @pl.loop(0, _CHW)
        def _main(i):
            do_chunk((wid * _CHW + i) * _CH)

        @pl.when(wid < _CHREM)
        def _tail():
            do_chunk((_NW * _CHW + wid) * _CH)

    return body(Pe, Qe, src, tgt)


def kernel(node_h, edge_index, edge_h, mW1, mb1, mW2, mb2, Wih, Whh, bih,
           bhh, gamma, beta, eW1, eb1, eW2, eb2):
    src = edge_index[0]
    tgt = edge_index[1]

    Wn = jnp.concatenate([mW1[:, :D].T, mW1[:, D:2 * D].T, Whh.T], axis=1)
    bn = jnp.concatenate([jnp.zeros((2 * D,), jnp.float32), bhh]).reshape(1, 5 * D)
    P, Q, GH = _node_pre(node_h, Wn, bn)

    Rm = _edge_pre(edge_h, mW1[:, 2 * D:].T, mb1.reshape(1, D))

    h1 = _sc_h1(P, Q, Rm, src, tgt)
    Hfull = jnp.zeros((_NPASS * _HALF, D), jnp.float32).at[tgt].add(h1)
    Dfull = jnp.zeros((_NPASS * _HALF, ED), jnp.float32).at[tgt].add(
        jnp.ones((E, ED), jnp.float32))
    Hr = jnp.stack([Hfull, jnp.zeros_like(Hfull)])
    Dr = jnp.stack([Dfull, jnp.zeros_like(Dfull)])

    eWst = jnp.concatenate([eW1[:, :D].T, eW1[:, D:2 * D].T], axis=1)
    normed, Pe, Qe = _node_upd(
        Hr[0, :N], Hr[1, :N], Dr[0, :N], Dr[1, :N], node_h, GH,
        mW2.T, mb2.reshape(1, D), Wih.T, bih.reshape(1, 3 * D),
        gamma.reshape(1, D), beta.reshape(1, D), eWst)

    esum = _sc_esum(Pe, Qe, src, tgt)

    new_edge = _edge_fin(esum, edge_h, eW1[:, 2 * D:].T, eb1.reshape(1, D),
                         eW2.T, eb2.reshape(1, ED))

    return normed, new_edge
